# baseline (speedup 1.0000x reference)
import functools

import jax
import jax.numpy as jnp
from jax import lax
from jax.experimental import pallas as pl
from jax.experimental.pallas import tpu as pltpu
from jax.experimental.pallas import tpu_sc as plsc

_NUM_FIELDS = 26
_FIELD_DIM = 100000
_TOTAL_ROWS = _NUM_FIELDS * _FIELD_DIM
_BATCH = 4096
_EMB = 16
_IN_DIM = _NUM_FIELDS * _EMB
_H_PAD = 512
_EPS_BN = 1e-5

_EROWS = _TOTAL_ROWS * _EMB // 128
_LROWS = (_TOTAL_ROWS + 127) // 128

_NSLOT = 3
_CHUNK = 64


def _sc_gather(X2, emb2, lin2):
    info = plsc.get_sparse_core_info()
    nc, ns = info.num_cores, info.num_subcores
    nw = nc * ns
    bpw = _BATCH // nw

    mesh = plsc.VectorSubcoreMesh(core_axis_name="c", subcore_axis_name="s")

    @functools.partial(
        pl.kernel,
        out_type=(
            jax.ShapeDtypeStruct((_H_PAD, _BATCH), jnp.float32),
            jax.ShapeDtypeStruct((_BATCH,), jnp.float32),
        ),
        mesh=mesh,
        compiler_params=pltpu.CompilerParams(needs_layout_passes=False),
        scratch_types=[
            pltpu.VMEM((32, bpw), jnp.int32),
            pltpu.VMEM((_NUM_FIELDS, bpw), jnp.int32),
            pltpu.VMEM((_NUM_FIELDS, bpw), jnp.int32),
            pltpu.VMEM((_NSLOT, _CHUNK, 128), jnp.float32),
            pltpu.VMEM((_NSLOT, _CHUNK, 128), jnp.float32),
            pltpu.VMEM((_H_PAD, bpw), jnp.float32),
            pltpu.VMEM((bpw,), jnp.float32),
            pltpu.SemaphoreType.DMA,
            pltpu.SemaphoreType.DMA,
        ],
    )
    def k(x_hbm, emb_hbm, lin_hbm, h_out, lin_out,
          x_v, erow_v, lrow_v, ebuf_v, lbuf_v, hblk_v,
          lacc_v, gsem, lsem):
        wid = lax.axis_index("s") * nc + lax.axis_index("c")
        b0 = wid * bpw
        pltpu.sync_copy(x_hbm.at[:, pl.ds(b0, bpw)], x_v)

        @pl.loop(0, _NUM_FIELDS)
        def _(f):
            @pl.loop(0, bpw, step=16)
            def _(g):
                idx = x_v[f, pl.ds(g, 16)] + f * _FIELD_DIM
                erow_v[f, pl.ds(g, 16)] = idx >> 3
                lrow_v[f, pl.ds(g, 16)] = idx >> 7

        @pl.loop(0, bpw, step=16)
        def _(g):
            lacc_v[pl.ds(g, 16)] = jnp.zeros((16,), jnp.float32)

        nround = (_NUM_FIELDS + _NSLOT - 1) // _NSLOT

        for half in range(bpw // _CHUNK):
            hb = half * _CHUNK

            for s in range(_NSLOT):
                pltpu.async_copy(
                    emb_hbm.at[erow_v.at[s, pl.ds(hb, _CHUNK)]],
                    ebuf_v.at[s], gsem)
                pltpu.async_copy(
                    lin_hbm.at[lrow_v.at[s, pl.ds(hb, _CHUNK)]],
                    lbuf_v.at[s], lsem)

            @pl.loop(0, nround)
            def _(r, hb=hb):
                for s in range(_NSLOT):
                    f = r * _NSLOT + s

                    @pl.when(f < _NUM_FIELDS)
                    def _(f=f, s=s, hb=hb):
                        pltpu.make_async_copy(
                            emb_hbm.at[erow_v.at[f, pl.ds(hb, _CHUNK)]],
                            ebuf_v.at[s], gsem).wait()

                        @pl.loop(0, _CHUNK, step=16)
                        def _(j0, f=f, s=s, hb=hb):
                            jv = (jnp.full((16,), j0, jnp.int32)
                                  + lax.iota(jnp.int32, 16))
                            idx = (x_v[f, pl.ds(hb + j0, 16)]
                                   + f * _FIELD_DIM)
                            off = (idx & 7) << 4
                            for e in range(_EMB):
                                vals = plsc.load_gather(
                                    ebuf_v.at[s], [jv, off + e])
                                hblk_v[f * _EMB + e,
                                       pl.ds(hb + j0, 16)] = vals

                        pltpu.make_async_copy(
                            lin_hbm.at[lrow_v.at[f, pl.ds(hb, _CHUNK)]],
                            lbuf_v.at[s], lsem).wait()

                        @pl.loop(0, _CHUNK, step=16)
                        def _(j0, f=f, s=s, hb=hb):
                            jv = (jnp.full((16,), j0, jnp.int32)
                                  + lax.iota(jnp.int32, 16))
                            idx = (x_v[f, pl.ds(hb + j0, 16)]
                                   + f * _FIELD_DIM)
                            loff = idx & 127
                            vals = plsc.load_gather(
                                lbuf_v.at[s], [jv, loff])
                            lacc_v[pl.ds(hb + j0, 16)] = (
                                lacc_v[pl.ds(hb + j0, 16)] + vals)

                        @pl.when(f + _NSLOT < _NUM_FIELDS)
                        def _(f=f, s=s, hb=hb):
                            pltpu.async_copy(
                                emb_hbm.at[
                                    erow_v.at[f + _NSLOT, pl.ds(hb, _CHUNK)]],
                                ebuf_v.at[s], gsem)
                            pltpu.async_copy(
                                lin_hbm.at[
                                    lrow_v.at[f + _NSLOT, pl.ds(hb, _CHUNK)]],
                                lbuf_v.at[s], lsem)

        pltpu.sync_copy(hblk_v, h_out.at[:, pl.ds(b0, bpw)])
        pltpu.sync_copy(lacc_v, lin_out.at[pl.ds(b0, bpw)])

    return k(X2, emb2, lin2)


def _tc_mlp(ht, lin, y, W1, b1, g1, bt1, W2, b2, g2, bt2, w3row, b3,
            lin_bias, S):
    def body(ht_ref, lin_ref, y_ref, w1_ref, b1_ref, g1_ref, bt1_ref,
             w2_ref, b2_ref, g2_ref, bt2_ref, w3_ref, b3_ref, lb_ref,
             s_ref, out_ref):
        ht_v = ht_ref[...]
        row = lax.broadcasted_iota(jnp.int32, ht_v.shape, 0)
        ht_v = jnp.where(row < _IN_DIM, ht_v, 0.0)
        dn0 = (((0,), (0,)), ((), ()))
        sm = s_ref[...]
        s = lax.dot_general(ht_v, sm, dn0,
                            preferred_element_type=jnp.float32)
        ss = lax.dot_general(ht_v * ht_v, sm, dn0,
                             preferred_element_type=jnp.float32)
        fm = 0.5 * jnp.sum(s * s - ss, axis=1, keepdims=True)

        z1 = lax.dot_general(ht_v, w1_ref[...], dn0,
                             preferred_element_type=jnp.float32)
        z1 = z1 + b1_ref[...]
        m1 = jnp.mean(z1, axis=0, keepdims=True)
        v1 = jnp.mean(z1 * z1, axis=0, keepdims=True) - m1 * m1
        a1 = jnp.maximum(
            g1_ref[...] * (z1 - m1) * lax.rsqrt(v1 + _EPS_BN) + bt1_ref[...],
            0.0)

        z2 = jnp.dot(a1, w2_ref[...], preferred_element_type=jnp.float32)
        z2 = z2 + b2_ref[...]
        m2 = jnp.mean(z2, axis=0, keepdims=True)
        v2 = jnp.mean(z2 * z2, axis=0, keepdims=True) - m2 * m2
        a2 = jnp.maximum(
            g2_ref[...] * (z2 - m2) * lax.rsqrt(v2 + _EPS_BN) + bt2_ref[...],
            0.0)

        z3 = jnp.sum(a2 * w3_ref[...], axis=1, keepdims=True) + b3_ref[...]
        logits = z3 + lin_ref[...] + lb_ref[...] + fm
        p = 1.0 / (1.0 + jnp.exp(-logits))
        p = jnp.clip(p, 1e-7, 1.0 - 1e-7)
        yv = y_ref[...]
        ll = yv * jnp.log(p) + (1.0 - yv) * jnp.log(1.0 - p)
        out_ref[...] = jnp.reshape(-jnp.sum(ll) * (1.0 / _BATCH), (1, 1))

    return pl.pallas_call(
        body,
        out_shape=jax.ShapeDtypeStruct((1, 1), jnp.float32),
    )(ht, lin, y, W1, b1, g1, bt1, W2, b2, g2, bt2, w3row, b3, lin_bias, S)


def kernel(X, y, emb_table, lin_table, lin_bias, W1, b1, g1, bt1,
           W2, b2, g2, bt2, W3, b3):
    X2 = jnp.pad(X, ((0, 32 - _NUM_FIELDS), (0, 0)))
    emb2 = emb_table.reshape(_EROWS, 128)
    lin2 = jnp.pad(lin_table.reshape(-1),
                   (0, _LROWS * 128 - _TOTAL_ROWS)).reshape(_LROWS, 128)
    ht, lin = _sc_gather(X2, emb2, lin2)

    S = jnp.pad(jnp.tile(jnp.eye(_EMB, dtype=jnp.float32), (_NUM_FIELDS, 1)),
                ((0, _H_PAD - _IN_DIM), (0, 0)))
    W1p = jnp.pad(W1, ((0, _H_PAD - _IN_DIM), (0, 0)))
    loss = _tc_mlp(
        ht, lin.reshape(_BATCH, 1), y,
        W1p, b1.reshape(1, -1), g1.reshape(1, -1), bt1.reshape(1, -1),
        W2, b2.reshape(1, -1), g2.reshape(1, -1), bt2.reshape(1, -1),
        W3.reshape(1, -1), b3.reshape(1, 1), lin_bias.reshape(1, 1), S)
    return loss[0, 0]
